# Initial kernel scaffold; baseline (speedup 1.0000x reference)
import jax
import jax.numpy as jnp
from jax.experimental import pallas as pl


def kernel(node_ids, emb, Wih, Whh, bih, bhh, sent_w, sent_b, ctx_w):
    raise NotImplementedError("write your pallas kernel here")


# speedup vs baseline: 4.3216x; 4.3216x over previous
import functools

import jax
import jax.numpy as jnp
from jax import lax
from jax.experimental import pallas as pl
from jax.experimental.pallas import tpu as pltpu
from jax.experimental.pallas import tpu_sc as plsc

_E = 512
_BS = 128
_N = 63
_LEAF0 = 31
_PADN = 64
_ROWS = _PADN * _BS

_NC, _NS = 2, 16
_NW = _NC * _NS
_BPW = _ROWS // _NW
_CH = 4
_CROWS = _BPW // _CH

@functools.cache
def _make_sc_gather():
    mesh = plsc.VectorSubcoreMesh(core_axis_name="c", subcore_axis_name="s")

    @functools.partial(
        pl.kernel,
        mesh=mesh,
        out_type=jax.ShapeDtypeStruct((_ROWS, _E), jnp.float32),
        scratch_types=[
            pltpu.VMEM((_CH, _CROWS), jnp.int32),
            pltpu.VMEM((_CROWS, _E), jnp.float32),
            pltpu.VMEM((_CROWS, _E), jnp.float32),
            pltpu.SemaphoreType.DMA,
            pltpu.SemaphoreType.DMA,
        ],
    )
    def _sc_gather(emb_hbm, idx_hbm, out_hbm, idx_v, buf0, buf1, sem0, sem1):
        wid = lax.axis_index("s") * _NC + lax.axis_index("c")
        base = wid * _BPW
        pltpu.sync_copy(idx_hbm.at[pl.ds(wid * _CH, _CH)], idx_v)
        bufs = (buf0, buf1)
        sems = (sem0, sem1)
        copies = [None, None]
        copies[0] = pltpu.async_copy(emb_hbm.at[idx_v.at[0]], buf0, sem0)
        for c in range(_CH):
            p = c & 1
            if c + 1 < _CH:
                q = (c + 1) & 1
                copies[q] = pltpu.async_copy(
                    emb_hbm.at[idx_v.at[c + 1]], bufs[q], sems[q])
            copies[p].wait()
            pltpu.sync_copy(bufs[p],
                            out_hbm.at[pl.ds(base + c * _CROWS, _CROWS)])

    return _sc_gather


def _tree_body(x_ref, wih_t_ref, whh_t_ref, bih_ref, bhh_ref, sw_ref, sb_ref,
               cw_ref, out_ref, h_all, macc):
    g = pl.program_id(0)
    node = _N - 1 - g
    row = node * _BS
    x = x_ref[pl.ds(row, _BS), :]
    gi = jnp.dot(x, wih_t_ref[:, :], preferred_element_type=jnp.float32)
    gi = gi + bih_ref[:, :]
    i_r = gi[:, :_E]
    i_z = gi[:, _E:2 * _E]
    i_n = gi[:, 2 * _E:]

    @pl.when(node >= _LEAF0)
    def _leaf():
        r = jax.nn.sigmoid(i_r + bhh_ref[:, :_E])
        z = jax.nn.sigmoid(i_z + bhh_ref[:, _E:2 * _E])
        n = jnp.tanh(i_n + r * bhh_ref[:, 2 * _E:])
        h_all[pl.ds(row, _BS), :] = (1.0 - z) * n

    @pl.when(node < _LEAF0)
    def _internal():
        c1 = 2 * node + 1
        h1 = h_all[pl.ds(c1 * _BS, _BS), :]
        h2 = h_all[pl.ds((c1 + 1) * _BS, _BS), :]
        u1 = jnp.tanh(jnp.dot(h1, sw_ref[:, :],
                              preferred_element_type=jnp.float32) + sb_ref[:, :])
        u2 = jnp.tanh(jnp.dot(h2, sw_ref[:, :],
                              preferred_element_type=jnp.float32) + sb_ref[:, :])
        s1 = jnp.tanh(jnp.sum(u1 * cw_ref[:, :], axis=1, keepdims=True))
        s2 = jnp.tanh(jnp.sum(u2 * cw_ref[:, :], axis=1, keepdims=True))
        w1 = jax.nn.sigmoid(s1 - s2)
        h0 = w1 * h1 + (1.0 - w1) * h2
        gh = jnp.dot(h0, whh_t_ref[:, :], preferred_element_type=jnp.float32)
        gh = gh + bhh_ref[:, :]
        r = jax.nn.sigmoid(i_r + gh[:, :_E])
        z = jax.nn.sigmoid(i_z + gh[:, _E:2 * _E])
        n = jnp.tanh(i_n + r * gh[:, 2 * _E:])
        h_all[pl.ds(row, _BS), :] = (1.0 - z) * n + z * h0

    h = h_all[pl.ds(row, _BS), :]

    @pl.when(g == 0)
    def _init():
        macc[:, :] = h

    @pl.when(g > 0)
    def _acc():
        macc[:, :] = jnp.maximum(macc[:, :], h)

    @pl.when(g == _N - 1)
    def _fin():
        out_ref[:, :] = macc[:, :]


def _tree_gru(x_pad, wih_t, whh_t, bih_r, bhh_r, sw, sb, cw_r):
    return pl.pallas_call(
        _tree_body,
        grid=(_N,),
        in_specs=[pl.BlockSpec(memory_space=pltpu.VMEM)] * 8,
        out_specs=pl.BlockSpec(memory_space=pltpu.VMEM),
        out_shape=jax.ShapeDtypeStruct((_BS, _E), jnp.float32),
        scratch_shapes=[
            pltpu.VMEM((_N * _BS, _E), jnp.float32),
            pltpu.VMEM((_BS, _E), jnp.float32),
        ],
        compiler_params=pltpu.CompilerParams(
            dimension_semantics=("arbitrary",)),
    )(x_pad, wih_t, whh_t, bih_r, bhh_r, sw, sb, cw_r)


def kernel(node_ids, emb, Wih, Whh, bih, bhh, sent_w, sent_b, ctx_w):
    ids = node_ids.reshape(-1).astype(jnp.int32)
    ids_pad = jnp.concatenate(
        [ids, jnp.zeros((_ROWS - _N * _BS,), jnp.int32)])
    idx2 = ids_pad.reshape(_NW * _CH, _CROWS)
    x_pad = _make_sc_gather()(emb, idx2)
    return _tree_gru(
        x_pad, Wih.T, Whh.T,
        bih.reshape(1, 3 * _E), bhh.reshape(1, 3 * _E),
        sent_w, sent_b, ctx_w.reshape(1, _E))
